# baseline (speedup 1.0000x reference)
import jax
import jax.numpy as jnp
from jax import lax
from jax.experimental import pallas as pl
from jax.experimental.pallas import tpu as pltpu
from jax.experimental.pallas import tpu_sc as plsc

N = 10000
E = 320000
D = 128
EPS = 1e-7

NC = 2
NS = 16
LK = 16

K = 80
CHUNKS = 252
EPT = CHUNKS * K
EPAD = EPT * NS
EXTRA = 2 * K
RPT = 632
NPAD = RPT * NS
RCHS = (80, 80, 80, 80, 80, 80, 80, 72)


def _norm(x):
    return jnp.maximum(jnp.sqrt(jnp.sum(x * x, axis=-1, keepdims=True)), EPS)


def _artanh(x):
    x = jnp.clip(x, -1.0 + 1e-6, 1.0 - 1e-6)
    return 0.5 * jnp.log((1.0 + x) / (1.0 - x))


def _mobius_scalar_mul(r, x):
    n = _norm(x)
    return jnp.tanh(r * _artanh(n)) * x / n


def _mobius_addition(x, y):
    xy = jnp.sum(x * y, axis=-1, keepdims=True)
    x2 = jnp.sum(x * x, axis=-1, keepdims=True)
    y2 = jnp.sum(y * y, axis=-1, keepdims=True)
    num = (1.0 + 2.0 * xy + y2) * x + (1.0 - x2) * y
    den = jnp.maximum(1.0 + 2.0 * xy + x2 * y2, EPS)
    return num / den


_ROWS_BLK = 2000


def _pre_body(ex_ref, lx_ref, tab_ref):
    i = pl.program_id(0)
    x = lx_ref[...]
    n = _norm(x)
    tan = _artanh(n) * x / n
    row = jnp.where(i < N // _ROWS_BLK, ex_ref[...], tan)
    row = row.reshape(_ROWS_BLK, D // 32, 2, LK)
    row = row.swapaxes(2, 3).reshape(_ROWS_BLK, D)
    tab_ref[...] = row.astype(jnp.bfloat16)


def _pre_tc(euclidean_x, lorentz_x):
    inb = pl.BlockSpec((_ROWS_BLK, D), lambda i: (i % (N // _ROWS_BLK), 0))
    return pl.pallas_call(
        _pre_body,
        out_shape=jax.ShapeDtypeStruct((2 * N, D), jnp.bfloat16),
        grid=(2 * N // _ROWS_BLK,),
        in_specs=[inb, inb],
        out_specs=pl.BlockSpec((_ROWS_BLK, D), lambda i: (i, 0)),
    )(euclidean_x, lorentz_x)


def _post_body(agge_ref, aggt_ref, ex_ref, lx_ref, eo_ref, lo_ref):
    eo_ref[...] = 0.5 * agge_ref[...] + 0.5 * ex_ref[...]
    t = aggt_ref[...]
    n = _norm(t)
    lorentz_pre = jnp.tanh(n) * t / n
    l_skip = _mobius_scalar_mul(0.5, lx_ref[...])
    l_out = _mobius_scalar_mul(0.5, lorentz_pre)
    lo_ref[...] = _mobius_addition(l_out, l_skip)


def _post_tc(agg_e, agg_t, euclidean_x, lorentz_x):
    blk = pl.BlockSpec((_ROWS_BLK, D), lambda i: (i, 0))
    return pl.pallas_call(
        _post_body,
        out_shape=(
            jax.ShapeDtypeStruct((N, D), jnp.float32),
            jax.ShapeDtypeStruct((N, D), jnp.float32),
        ),
        grid=(N // _ROWS_BLK,),
        in_specs=[blk, blk, blk, blk],
        out_specs=(blk, blk),
    )(agg_e, agg_t, euclidean_x, lorentz_x)


def _sc_spmm(xcat_bf, src_all, dst, val):
    mesh = plsc.VectorSubcoreMesh(
        core_axis_name="c", subcore_axis_name="s", num_cores=NC, num_subcores=NS
    )

    def body(xcat_hbm, src_hbm, dst_hbm, val_hbm, out_hbm,
             src_a, src_b, src_c, dst_a, dst_b, dst_c, val_a, val_b, val_c,
             rin_a, rin_b, rin_c, rout_a, rout_b, rout_c, acc_sh,
             sem_ia, sem_ib, sem_ic, sem_ga, sem_gb, sem_gc,
             sem_sa, sem_sb, sem_sc):
        c = lax.axis_index("c")
        s = lax.axis_index("s")
        zero16f = jnp.zeros((LK,), jnp.float32)
        himask = jnp.full((LK,), -65536, jnp.int32)

        sets = (
            (src_a, dst_a, val_a, rin_a, rout_a, sem_ia, sem_ga, sem_sa),
            (src_b, dst_b, val_b, rin_b, rout_b, sem_ib, sem_gb, sem_sb),
            (src_c, dst_c, val_c, rin_c, rout_c, sem_ic, sem_gc, sem_sc),
        )

        def idx_load(g, st):
            src_v, dst_v, val_v, _, _, sem, _, _ = st
            e0 = s * EPT + g * K
            pltpu.async_copy(src_hbm.at[pl.ds(c * (EPAD + EXTRA) + e0, K)],
                             src_v, sem)
            pltpu.async_copy(dst_hbm.at[pl.ds(e0, K)], dst_v, sem)
            pltpu.async_copy(val_hbm.at[pl.ds(e0, K)], val_v, sem)

        def idx_wait(st):
            src_v, dst_v, val_v, _, _, sem, _, _ = st
            pltpu.make_async_copy(src_hbm.at[pl.ds(0, K)], src_v, sem).wait()
            pltpu.make_async_copy(dst_hbm.at[pl.ds(0, K)], dst_v, sem).wait()
            pltpu.make_async_copy(val_hbm.at[pl.ds(0, K)], val_v, sem).wait()

        def gather(st):
            src_v, _, _, rin, _, _, sem, _ = st
            pltpu.async_copy(xcat_hbm.at[src_v], rin, sem)

        def wait_gather(st):
            _, _, _, rin, _, _, sem, _ = st
            pltpu.make_async_copy(xcat_hbm.at[pl.ds(0, K)], rin, sem).wait()

        def scatter(st):
            _, dst_v, _, _, rout, _, _, sem = st
            pltpu.async_copy(rout, acc_sh.at[dst_v], sem, add=True)

        def wait_scatter(st):
            _, dst_v, _, _, rout, _, _, sem = st
            pltpu.make_async_copy(rout, acc_sh.at[dst_v], sem).wait()

        def scale(st):
            _, _, val_v, rin, rout, _, _, _ = st

            @plsc.parallel_loop(0, K // LK)
            def grp(t):
                vals16 = val_v[pl.ds(t * LK, LK)]
                for el in range(LK):
                    e = t * LK + el
                    v = vals16[el]
                    for j in range(D // 32):
                        w32 = rin[e, pl.ds(j * 32, 32)]
                        w = plsc.bitcast(w32, jnp.int32)
                        lo = plsc.bitcast(
                            lax.shift_left(w, 16), jnp.float32)
                        hi = plsc.bitcast(
                            jnp.bitwise_and(w, himask), jnp.float32)
                        rout[e, pl.ds(j * 32, LK)] = lo * v
                        rout[e, pl.ds(j * 32 + LK, LK)] = hi * v

        def zrow(r, carry):
            for j in range(D // LK):
                rout_a[r, pl.ds(j * LK, LK)] = zero16f
            return carry

        lax.fori_loop(0, K, zrow, 0)
        off = 0
        for sz in RCHS:
            pltpu.sync_copy(
                rout_a.at[pl.ds(0, sz)],
                acc_sh.at[pl.ds(s * RPT + off, sz)],
            )
            off += sz
        plsc.subcore_barrier()

        idx_load(jnp.int32(0), sets[0])
        idx_load(jnp.int32(1), sets[1])
        idx_wait(sets[0])
        gather(sets[0])
        idx_wait(sets[1])
        gather(sets[1])

        def step(m, carry):
            for r in range(3):
                g = 3 * m + r
                stx = sets[r]
                stq = sets[(r + 2) % 3]
                if r == 0:
                    @pl.when(m > 0)
                    def _():
                        wait_scatter(stq)
                else:
                    wait_scatter(stq)
                idx_load(g + 2, stq)
                wait_gather(stx)
                scale(stx)
                scatter(stx)
                idx_wait(stq)
                gather(stq)
            return carry

        lax.fori_loop(0, CHUNKS // 3, step, 0)
        wait_gather(sets[CHUNKS % 3])
        wait_gather(sets[(CHUNKS + 1) % 3])
        wait_scatter(sets[(CHUNKS - 1) % 3])
        plsc.subcore_barrier()

        off = 0
        for sz in RCHS:
            pltpu.sync_copy(
                acc_sh.at[pl.ds(s * RPT + off, sz)],
                out_hbm.at[pl.ds(c * NPAD + s * RPT + off, sz)],
            )
            off += sz

    f = pl.kernel(
        body,
        out_type=jax.ShapeDtypeStruct((NC * NPAD, D), jnp.float32),
        mesh=mesh,
        compiler_params=pltpu.CompilerParams(
            needs_layout_passes=False, use_tc_tiling_on_sc=False),
        scratch_types=(
            [pltpu.VMEM((K,), jnp.int32)] * 3
            + [pltpu.VMEM((K,), jnp.int32)] * 3
            + [pltpu.VMEM((K,), jnp.float32)] * 3
            + [pltpu.VMEM((K, D), jnp.bfloat16)] * 3
            + [pltpu.VMEM((K, D), jnp.float32)] * 3
            + [pltpu.VMEM_SHARED((NPAD, D), jnp.float32)]
            + [pltpu.SemaphoreType.DMA] * 9
        ),
    )
    return f(xcat_bf, src_all, dst, val)


def kernel(euclidean_x, lorentz_x, adj_indices, adj_values):
    xcat_bf = _pre_tc(euclidean_x, lorentz_x)
    pad = EPAD + EXTRA - E
    dst = jnp.concatenate([adj_indices[0], jnp.zeros((pad,), jnp.int32)])
    src = jnp.concatenate([adj_indices[1], jnp.zeros((pad,), jnp.int32)])
    val = jnp.concatenate([adj_values, jnp.zeros((pad,), jnp.float32)])
    src_all = jnp.concatenate([src, src + N])
    agg = _sc_spmm(xcat_bf, src_all, dst, val)
    return _post_tc(agg[:N], agg[NPAD:NPAD + N], euclidean_x, lorentz_x)


# speedup vs baseline: 1.4925x; 1.4925x over previous
import jax
import jax.numpy as jnp
from jax import lax
from jax.experimental import pallas as pl
from jax.experimental.pallas import tpu as pltpu
from jax.experimental.pallas import tpu_sc as plsc

N = 10000
E = 320000
D = 128
EPS = 1e-7

NC = 2
NS = 16
LK = 16

K = 80
CHUNKS = 252
EPT = CHUNKS * K
EPAD = EPT * NS
EXTRA = 2 * K
RPT = 632
NPAD = RPT * NS
RCHS = (80, 80, 80, 80, 80, 80, 80, 72)


def _norm(x):
    return jnp.maximum(jnp.sqrt(jnp.sum(x * x, axis=-1, keepdims=True)), EPS)


def _artanh(x):
    x = jnp.clip(x, -1.0 + 1e-6, 1.0 - 1e-6)
    return 0.5 * jnp.log((1.0 + x) / (1.0 - x))


def _mobius_scalar_mul(r, x):
    n = _norm(x)
    return jnp.tanh(r * _artanh(n)) * x / n


def _mobius_addition(x, y):
    xy = jnp.sum(x * y, axis=-1, keepdims=True)
    x2 = jnp.sum(x * x, axis=-1, keepdims=True)
    y2 = jnp.sum(y * y, axis=-1, keepdims=True)
    num = (1.0 + 2.0 * xy + y2) * x + (1.0 - x2) * y
    den = jnp.maximum(1.0 + 2.0 * xy + x2 * y2, EPS)
    return num / den


_ROWS_BLK = 2000


def _pre_body(lx_ref, tan_ref):
    x = lx_ref[...]
    n = _norm(x)
    tan_ref[...] = _artanh(n) * x / n


def _pre_tc(lorentz_x):
    return pl.pallas_call(
        _pre_body,
        out_shape=jax.ShapeDtypeStruct((N, D), jnp.float32),
        grid=(N // _ROWS_BLK,),
        in_specs=[pl.BlockSpec((_ROWS_BLK, D), lambda i: (i, 0))],
        out_specs=pl.BlockSpec((_ROWS_BLK, D), lambda i: (i, 0)),
    )(lorentz_x)


def _pack_table(x):
    n = x.shape[0]
    xi = x.reshape(n, D // 32, 2, LK).transpose(0, 1, 3, 2)
    return xi.astype(jnp.bfloat16).reshape(n, D)


def _post_body(agge_ref, aggt_ref, ex_ref, lx_ref, eo_ref, lo_ref):
    eo_ref[...] = 0.5 * agge_ref[...] + 0.5 * ex_ref[...]
    t = aggt_ref[...]
    n = _norm(t)
    lorentz_pre = jnp.tanh(n) * t / n
    l_skip = _mobius_scalar_mul(0.5, lx_ref[...])
    l_out = _mobius_scalar_mul(0.5, lorentz_pre)
    lo_ref[...] = _mobius_addition(l_out, l_skip)


def _post_tc(agg_e, agg_t, euclidean_x, lorentz_x):
    blk = pl.BlockSpec((_ROWS_BLK, D), lambda i: (i, 0))
    return pl.pallas_call(
        _post_body,
        out_shape=(
            jax.ShapeDtypeStruct((N, D), jnp.float32),
            jax.ShapeDtypeStruct((N, D), jnp.float32),
        ),
        grid=(N // _ROWS_BLK,),
        in_specs=[blk, blk, blk, blk],
        out_specs=(blk, blk),
    )(agg_e, agg_t, euclidean_x, lorentz_x)


def _sc_spmm(xcat_bf, src_all, dst, val):
    mesh = plsc.VectorSubcoreMesh(
        core_axis_name="c", subcore_axis_name="s", num_cores=NC, num_subcores=NS
    )

    def body(xcat_hbm, src_hbm, dst_hbm, val_hbm, out_hbm,
             src_a, src_b, src_c, dst_a, dst_b, dst_c, val_a, val_b, val_c,
             rin_a, rin_b, rin_c, rout_a, rout_b, rout_c, acc_sh,
             sem_ia, sem_ib, sem_ic, sem_ga, sem_gb, sem_gc,
             sem_sa, sem_sb, sem_sc):
        c = lax.axis_index("c")
        s = lax.axis_index("s")
        zero16f = jnp.zeros((LK,), jnp.float32)
        himask = jnp.full((LK,), -65536, jnp.int32)

        sets = (
            (src_a, dst_a, val_a, rin_a, rout_a, sem_ia, sem_ga, sem_sa),
            (src_b, dst_b, val_b, rin_b, rout_b, sem_ib, sem_gb, sem_sb),
            (src_c, dst_c, val_c, rin_c, rout_c, sem_ic, sem_gc, sem_sc),
        )

        def idx_load(g, st):
            src_v, dst_v, val_v, _, _, sem, _, _ = st
            e0 = s * EPT + g * K
            pltpu.async_copy(src_hbm.at[pl.ds(c * (EPAD + EXTRA) + e0, K)],
                             src_v, sem)
            pltpu.async_copy(dst_hbm.at[pl.ds(e0, K)], dst_v, sem)
            pltpu.async_copy(val_hbm.at[pl.ds(e0, K)], val_v, sem)

        def idx_wait(st):
            src_v, dst_v, val_v, _, _, sem, _, _ = st
            pltpu.make_async_copy(src_hbm.at[pl.ds(0, K)], src_v, sem).wait()
            pltpu.make_async_copy(dst_hbm.at[pl.ds(0, K)], dst_v, sem).wait()
            pltpu.make_async_copy(val_hbm.at[pl.ds(0, K)], val_v, sem).wait()

        def gather(st):
            src_v, _, _, rin, _, _, sem, _ = st
            pltpu.async_copy(xcat_hbm.at[src_v], rin, sem)

        def wait_gather(st):
            _, _, _, rin, _, _, sem, _ = st
            pltpu.make_async_copy(xcat_hbm.at[pl.ds(0, K)], rin, sem).wait()

        def scatter(st):
            _, dst_v, _, _, rout, _, _, sem = st
            pltpu.async_copy(rout, acc_sh.at[dst_v], sem, add=True)

        def wait_scatter(st):
            _, dst_v, _, _, rout, _, _, sem = st
            pltpu.make_async_copy(rout, acc_sh.at[dst_v], sem).wait()

        def scale(st):
            _, _, val_v, rin, rout, _, _, _ = st

            @plsc.parallel_loop(0, K // LK)
            def grp(t):
                vals16 = val_v[pl.ds(t * LK, LK)]
                for el in range(LK):
                    e = t * LK + el
                    v = vals16[el]
                    for j in range(D // 32):
                        w32 = rin[e, pl.ds(j * 32, 32)]
                        w = plsc.bitcast(w32, jnp.int32)
                        lo = plsc.bitcast(
                            lax.shift_left(w, 16), jnp.float32)
                        hi = plsc.bitcast(
                            jnp.bitwise_and(w, himask), jnp.float32)
                        rout[e, pl.ds(j * 32, LK)] = lo * v
                        rout[e, pl.ds(j * 32 + LK, LK)] = hi * v

        def zrow(r, carry):
            for j in range(D // LK):
                rout_a[r, pl.ds(j * LK, LK)] = zero16f
            return carry

        lax.fori_loop(0, K, zrow, 0)
        off = 0
        for sz in RCHS:
            pltpu.sync_copy(
                rout_a.at[pl.ds(0, sz)],
                acc_sh.at[pl.ds(s * RPT + off, sz)],
            )
            off += sz
        plsc.subcore_barrier()

        idx_load(jnp.int32(0), sets[0])
        idx_load(jnp.int32(1), sets[1])
        idx_wait(sets[0])
        gather(sets[0])
        idx_wait(sets[1])
        gather(sets[1])

        def step(m, carry):
            for r in range(3):
                g = 3 * m + r
                stx = sets[r]
                stq = sets[(r + 2) % 3]
                if r == 0:
                    @pl.when(m > 0)
                    def _():
                        wait_scatter(stq)
                else:
                    wait_scatter(stq)
                idx_load(g + 2, stq)
                wait_gather(stx)
                scale(stx)
                scatter(stx)
                idx_wait(stq)
                gather(stq)
            return carry

        lax.fori_loop(0, CHUNKS // 3, step, 0)
        wait_gather(sets[CHUNKS % 3])
        wait_gather(sets[(CHUNKS + 1) % 3])
        wait_scatter(sets[(CHUNKS - 1) % 3])
        plsc.subcore_barrier()

        off = 0
        for sz in RCHS:
            pltpu.sync_copy(
                acc_sh.at[pl.ds(s * RPT + off, sz)],
                out_hbm.at[pl.ds(c * NPAD + s * RPT + off, sz)],
            )
            off += sz

    f = pl.kernel(
        body,
        out_type=jax.ShapeDtypeStruct((NC * NPAD, D), jnp.float32),
        mesh=mesh,
        compiler_params=pltpu.CompilerParams(
            needs_layout_passes=False, use_tc_tiling_on_sc=False),
        scratch_types=(
            [pltpu.VMEM((K,), jnp.int32)] * 3
            + [pltpu.VMEM((K,), jnp.int32)] * 3
            + [pltpu.VMEM((K,), jnp.float32)] * 3
            + [pltpu.VMEM((K, D), jnp.bfloat16)] * 3
            + [pltpu.VMEM((K, D), jnp.float32)] * 3
            + [pltpu.VMEM_SHARED((NPAD, D), jnp.float32)]
            + [pltpu.SemaphoreType.DMA] * 9
        ),
    )
    return f(xcat_bf, src_all, dst, val)


def kernel(euclidean_x, lorentz_x, adj_indices, adj_values):
    tangent_x = _pre_tc(lorentz_x)
    xcat_bf = jnp.concatenate(
        [_pack_table(euclidean_x), _pack_table(tangent_x)], axis=0)
    pad = EPAD + EXTRA - E
    dst = jnp.concatenate([adj_indices[0], jnp.zeros((pad,), jnp.int32)])
    src = jnp.concatenate([adj_indices[1], jnp.zeros((pad,), jnp.int32)])
    val = jnp.concatenate([adj_values, jnp.zeros((pad,), jnp.float32)])
    src_all = jnp.concatenate([src, src + N])
    agg = _sc_spmm(xcat_bf, src_all, dst, val)
    return _post_tc(agg[:N], agg[NPAD:NPAD + N], euclidean_x, lorentz_x)
